# Initial kernel scaffold; baseline (speedup 1.0000x reference)
import jax
import jax.numpy as jnp
from jax.experimental import pallas as pl


def kernel(x, Wg, W1, b1, W2, b2):
    raise NotImplementedError("write your pallas kernel here")


# speedup vs baseline: 1.1064x; 1.1064x over previous
import functools

import jax
import jax.numpy as jnp
from jax.experimental import pallas as pl
from jax.experimental.pallas import tpu as pltpu


def _moe_dense_kernel(x_ref, wg_ref, w1_ref, b1_ref, w2_ref, b2_ref,
                      out_ref, comb_ref, *, n_experts):
    t = pl.program_id(0)
    e = pl.program_id(1)
    S = x_ref.shape[0]

    @pl.when(e == 0)
    def _router():
        logits = jax.lax.dot_general(
            x_ref[...], wg_ref[...],
            (((1,), (1,)), ((), ())),
            preferred_element_type=jnp.float32)
        m = jnp.max(logits, axis=-1, keepdims=True)
        p = jnp.exp(logits - m)
        p = p / jnp.sum(p, axis=-1, keepdims=True)
        lane = jax.lax.broadcasted_iota(jnp.int32, p.shape, 1)
        m1 = jnp.max(p, axis=-1, keepdims=True)
        i1 = jnp.min(jnp.where(p == m1, lane, n_experts), axis=-1,
                     keepdims=True)
        mask1 = lane == i1
        p2 = jnp.where(mask1, -jnp.inf, p)
        m2 = jnp.max(p2, axis=-1, keepdims=True)
        i2 = jnp.min(jnp.where(p2 == m2, lane, n_experts), axis=-1,
                     keepdims=True)
        mask2 = lane == i2
        denom = m1 + m2
        comb = jnp.where(mask1, m1, 0.0) + jnp.where(mask2, m2, 0.0)
        comb_ref[...] = comb / denom

    h = jax.lax.dot_general(
        x_ref[...].astype(jnp.bfloat16), w1_ref[0],
        (((1,), (1,)), ((), ())),
        preferred_element_type=jnp.float32)
    h = h + b1_ref[0]
    h = h * jax.nn.sigmoid(h)
    oe = jax.lax.dot_general(
        h.astype(jnp.bfloat16), w2_ref[0],
        (((1,), (1,)), ((), ())),
        preferred_element_type=jnp.float32)
    oe = oe + b2_ref[0]

    lane = jax.lax.broadcasted_iota(jnp.int32, comb_ref.shape, 1)
    ce = jnp.sum(jnp.where(lane == e, comb_ref[...], 0.0), axis=-1,
                 keepdims=True)
    contrib = oe * ce

    @pl.when(e == 0)
    def _init():
        out_ref[...] = contrib

    @pl.when(e != 0)
    def _acc():
        out_ref[...] = out_ref[...] + contrib


def kernel(x, Wg, W1, b1, W2, b2):
    B, S, H = x.shape
    E, F, _ = W1.shape
    TS = 1024
    NT = S // TS

    xs = x[0]
    wg = Wg
    w1 = W1.astype(jnp.bfloat16)
    w2 = W2.astype(jnp.bfloat16)

    out = pl.pallas_call(
        functools.partial(_moe_dense_kernel, n_experts=E),
        grid=(NT, E),
        in_specs=[
            pl.BlockSpec((TS, H), lambda t, e: (t, 0)),
            pl.BlockSpec((E, H), lambda t, e: (0, 0)),
            pl.BlockSpec((1, F, H), lambda t, e: (e, 0, 0)),
            pl.BlockSpec((1, 1, F), lambda t, e: (e, 0, 0)),
            pl.BlockSpec((1, H, F), lambda t, e: (e, 0, 0)),
            pl.BlockSpec((1, 1, H), lambda t, e: (e, 0, 0)),
        ],
        out_specs=pl.BlockSpec((TS, H), lambda t, e: (t, 0)),
        out_shape=jax.ShapeDtypeStruct((S, H), jnp.float32),
        scratch_shapes=[pltpu.VMEM((TS, E), jnp.float32)],
    )(xs, wg, w1, b1.reshape(E, 1, F), w2, b2.reshape(E, 1, H))
    return out[None]
